# baseline (speedup 1.0000x reference)
import functools

import jax
import jax.numpy as jnp
from jax.experimental import pallas as pl
from jax.experimental.pallas import tpu as pltpu

_LANE = 128
_TILE_ROWS = 8192


def _pf_packed_kernel(xp_ref, m_ref, u_ref, b_ref, o_ref):
    z = xp_ref[...]
    arg = jax.lax.dot(
        z.astype(jnp.bfloat16), m_ref[...],
        preferred_element_type=jnp.float32) + b_ref[0]
    o_ref[...] = z + u_ref[...] * jnp.tanh(arg)


def _pf_rows_kernel(x_ref, w_ref, u_ref, b_ref, o_ref):
    z = x_ref[...]
    arg = jnp.sum(z * w_ref[...], axis=-1, keepdims=True) + b_ref[0]
    o_ref[...] = z + u_ref[...] * jnp.tanh(arg)


@functools.partial(jax.jit, static_argnames=("tile_rows",))
def _planar_flow(x, w, u, b, tile_rows=_TILE_ROWS):
    N, d = x.shape
    b_f32 = b.reshape(1).astype(jnp.float32)

    if d <= _LANE and _LANE % d == 0 and (N * d) % _LANE == 0:
        k = _LANE // d
        R = N // k
        P = _LANE
        xp = x.reshape(R, P)

        idx = jnp.arange(P)
        grp = idx // d
        wvec = w.reshape(-1).astype(jnp.float32)[idx % d]
        m = jnp.where(grp[:, None] == grp[None, :], wvec[:, None], 0.0)
        m_bf16 = m.astype(jnp.bfloat16)
        u_packed = jnp.tile(u.reshape(1, d).astype(jnp.float32), (1, k))

        tr = min(tile_rows, R)
        out = pl.pallas_call(
            _pf_packed_kernel,
            out_shape=jax.ShapeDtypeStruct((R, P), x.dtype),
            grid=(pl.cdiv(R, tr),),
            in_specs=[
                pl.BlockSpec((tr, P), lambda i: (i, 0)),
                pl.BlockSpec((P, P), lambda i: (0, 0)),
                pl.BlockSpec((1, P), lambda i: (0, 0)),
                pl.BlockSpec(memory_space=pltpu.MemorySpace.SMEM),
            ],
            out_specs=pl.BlockSpec((tr, P), lambda i: (i, 0)),
            compiler_params=pltpu.CompilerParams(
                dimension_semantics=("parallel",),
            ),
        )(xp, m_bf16, u_packed, b_f32)
        return out.reshape(N, d)

    w_f32 = w.reshape(1, d).astype(jnp.float32)
    u_f32 = u.reshape(1, d).astype(jnp.float32)
    tn = min(tile_rows, N)
    return pl.pallas_call(
        _pf_rows_kernel,
        out_shape=jax.ShapeDtypeStruct((N, d), x.dtype),
        grid=(pl.cdiv(N, tn),),
        in_specs=[
            pl.BlockSpec((tn, d), lambda i: (i, 0)),
            pl.BlockSpec((1, d), lambda i: (0, 0)),
            pl.BlockSpec((1, d), lambda i: (0, 0)),
            pl.BlockSpec(memory_space=pltpu.MemorySpace.SMEM),
        ],
        out_specs=pl.BlockSpec((tn, d), lambda i: (i, 0)),
        compiler_params=pltpu.CompilerParams(
            dimension_semantics=("parallel",),
        ),
    )(x, w_f32, u_f32, b_f32)


def kernel(x, w, u, b):
    return _planar_flow(x, w, u, b)


# speedup vs baseline: 1.4411x; 1.4411x over previous
import functools

import jax
import jax.numpy as jnp
from jax.experimental import pallas as pl
from jax.experimental.pallas import tpu as pltpu

_TILE_ROWS = 8192


def _pf_rows_kernel(x_ref, w_ref, u_ref, b_ref, o_ref):
    z = x_ref[...]
    arg = jnp.sum(z * w_ref[...], axis=-1, keepdims=True) + b_ref[0]
    o_ref[...] = z + u_ref[...] * jnp.tanh(arg)


@functools.partial(jax.jit, static_argnames=("tile_rows",))
def _planar_flow(x, w, u, b, tile_rows=_TILE_ROWS):
    N, d = x.shape
    b_f32 = b.reshape(1).astype(jnp.float32)
    w_f32 = w.reshape(1, d).astype(jnp.float32)
    u_f32 = u.reshape(1, d).astype(jnp.float32)
    tn = min(tile_rows, N)
    return pl.pallas_call(
        _pf_rows_kernel,
        out_shape=jax.ShapeDtypeStruct((N, d), x.dtype),
        grid=(pl.cdiv(N, tn),),
        in_specs=[
            pl.BlockSpec((tn, d), lambda i: (i, 0)),
            pl.BlockSpec((1, d), lambda i: (0, 0)),
            pl.BlockSpec((1, d), lambda i: (0, 0)),
            pl.BlockSpec(memory_space=pltpu.MemorySpace.SMEM),
        ],
        out_specs=pl.BlockSpec((tn, d), lambda i: (i, 0)),
        compiler_params=pltpu.CompilerParams(
            dimension_semantics=("parallel",),
        ),
    )(x, w_f32, u_f32, b_f32)


def kernel(x, w, u, b):
    return _planar_flow(x, w, u, b)


# speedup vs baseline: 1.4507x; 1.0067x over previous
import functools

import jax
import jax.numpy as jnp
from jax.experimental import pallas as pl
from jax.experimental.pallas import tpu as pltpu

_TILE_ROWS = 16384


def _pf_rows_kernel(x_ref, w_ref, u_ref, b_ref, o_ref):
    z = x_ref[...]
    arg = jnp.sum(z * w_ref[...], axis=-1, keepdims=True) + b_ref[0]
    o_ref[...] = z + u_ref[...] * jnp.tanh(arg)


@functools.partial(jax.jit, static_argnames=("tile_rows",))
def _planar_flow(x, w, u, b, tile_rows=_TILE_ROWS):
    N, d = x.shape
    b_f32 = b.reshape(1).astype(jnp.float32)
    w_f32 = w.reshape(1, d).astype(jnp.float32)
    u_f32 = u.reshape(1, d).astype(jnp.float32)
    tn = min(tile_rows, N)
    return pl.pallas_call(
        _pf_rows_kernel,
        out_shape=jax.ShapeDtypeStruct((N, d), x.dtype),
        grid=(pl.cdiv(N, tn),),
        in_specs=[
            pl.BlockSpec((tn, d), lambda i: (i, 0)),
            pl.BlockSpec((1, d), lambda i: (0, 0)),
            pl.BlockSpec((1, d), lambda i: (0, 0)),
            pl.BlockSpec(memory_space=pltpu.MemorySpace.SMEM),
        ],
        out_specs=pl.BlockSpec((tn, d), lambda i: (i, 0)),
        compiler_params=pltpu.CompilerParams(
            dimension_semantics=("parallel",),
        ),
    )(x, w_f32, u_f32, b_f32)


def kernel(x, w, u, b):
    return _planar_flow(x, w, u, b)


# speedup vs baseline: 7.2903x; 5.0253x over previous
import functools

import jax
import jax.numpy as jnp
from jax.experimental import pallas as pl
from jax.experimental.pallas import tpu as pltpu

_TILE_LANES = 16384


def _pf_t_kernel(xt_ref, w_ref, u_ref, b_ref, o_ref):
    z = xt_ref[...]
    zb = z.astype(jnp.bfloat16)
    s = jax.lax.dot_general(
        w_ref[...], zb, (((1,), (0,)), ((), ())),
        preferred_element_type=jnp.float32)
    t = jnp.tanh(s + b_ref[0]).astype(jnp.bfloat16)
    o_ref[...] = z + jax.lax.dot_general(
        u_ref[...], t, (((1,), (0,)), ((), ())),
        preferred_element_type=jnp.float32)


@functools.partial(jax.jit, static_argnames=("tile_lanes",))
def _planar_flow(x, w, u, b, tile_lanes=_TILE_LANES):
    N, d = x.shape
    xt = x.T
    w_bf = w.reshape(1, d).astype(jnp.bfloat16)
    u_col = u.reshape(1, d).astype(jnp.bfloat16).T
    b_f32 = b.reshape(1).astype(jnp.float32)
    tl = min(tile_lanes, N)
    out_t = pl.pallas_call(
        _pf_t_kernel,
        out_shape=jax.ShapeDtypeStruct((d, N), x.dtype),
        grid=(pl.cdiv(N, tl),),
        in_specs=[
            pl.BlockSpec((d, tl), lambda i: (0, i)),
            pl.BlockSpec((1, d), lambda i: (0, 0)),
            pl.BlockSpec((d, 1), lambda i: (0, 0)),
            pl.BlockSpec(memory_space=pltpu.MemorySpace.SMEM),
        ],
        out_specs=pl.BlockSpec((d, tl), lambda i: (0, i)),
        compiler_params=pltpu.CompilerParams(
            dimension_semantics=("parallel",),
        ),
    )(xt, w_bf, u_col, b_f32)
    return out_t.T


def kernel(x, w, u, b):
    return _planar_flow(x, w, u, b)


# speedup vs baseline: 8.0735x; 1.1074x over previous
import functools

import jax
import jax.numpy as jnp
from jax.experimental import pallas as pl
from jax.experimental.pallas import tpu as pltpu

_TILE_LANES = 16384


def _pf_t_kernel(xt_ref, w_ref, u_ref, b_ref, o_ref):
    z = xt_ref[...]
    s = jax.lax.dot_general(
        w_ref[...].astype(jnp.bfloat16), z.astype(jnp.bfloat16),
        (((1,), (0,)), ((), ())),
        preferred_element_type=jnp.float32)
    t = jnp.tanh(s + b_ref[0]).astype(jnp.bfloat16)
    o_ref[...] = z + jax.lax.dot_general(
        u_ref[...].astype(jnp.bfloat16), t,
        (((0,), (0,)), ((), ())),
        preferred_element_type=jnp.float32)


@functools.partial(jax.jit, static_argnames=("tile_lanes",))
def _planar_flow(x, w, u, b, tile_lanes=_TILE_LANES):
    N, d = x.shape
    xt = x.T
    tl = min(tile_lanes, N)
    out_t = pl.pallas_call(
        _pf_t_kernel,
        out_shape=jax.ShapeDtypeStruct((d, N), x.dtype),
        grid=(pl.cdiv(N, tl),),
        in_specs=[
            pl.BlockSpec((d, tl), lambda i: (0, i)),
            pl.BlockSpec((1, d), lambda i: (0, 0)),
            pl.BlockSpec((1, d), lambda i: (0, 0)),
            pl.BlockSpec(memory_space=pltpu.MemorySpace.SMEM),
        ],
        out_specs=pl.BlockSpec((d, tl), lambda i: (0, i)),
        compiler_params=pltpu.CompilerParams(
            dimension_semantics=("parallel",),
        ),
    )(xt, w.reshape(1, d), u.reshape(1, d), b.reshape(1))
    return out_t.T


def kernel(x, w, u, b):
    return _planar_flow(x, w, u, b)
